# baseline (speedup 1.0000x reference)
import jax
import jax.numpy as jnp
from jax.experimental import pallas as pl


def _tc_loss(v, up, un):
    def body(v_ref, p_ref, n_ref, o_ref):
        ps = jnp.sum(v_ref[...] * p_ref[...], axis=1)
        ns = jnp.sum(v_ref[...] * n_ref[...], axis=1)
        sp = jax.nn.sigmoid(ps)
        sn = jax.nn.sigmoid(ns)
        loss = (-jnp.mean(jnp.log(sp + 1e-09))
                - jnp.mean(jnp.log(1.0 - sn + 1e-09)))
        o_ref[...] = jnp.broadcast_to(loss, (1, 1))

    out = pl.pallas_call(
        body,
        out_shape=jax.ShapeDtypeStruct((1, 1), jnp.float32),
    )(v, up, un)
    return out[0, 0]


def kernel(center, pos, neg, input_emb, output_emb):
    v = jnp.take(input_emb, center, axis=0)
    up = jnp.take(output_emb, pos, axis=0)
    un = jnp.take(output_emb, neg, axis=0)
    return _tc_loss(v, up, un)
